# Initial kernel scaffold; baseline (speedup 1.0000x reference)
import jax
import jax.numpy as jnp
from jax.experimental import pallas as pl


def kernel(x, W_s, W_t):
    raise NotImplementedError("write your pallas kernel here")


# speedup vs baseline: 2.2264x; 2.2264x over previous
import functools

import jax
import jax.numpy as jnp
from jax import lax
from jax.experimental import pallas as pl
from jax.experimental.pallas import tpu as pltpu
from jax.experimental.pallas import tpu_sc as plsc

B = 16384
F_DENSE = 64
DIM_S = 32
DIM_T = 32
NC = 2
NS = 16
NW = NC * NS
ROWS_PER_W = B // NW
IDX_CHUNK = 128
N_CHUNKS = ROWS_PER_W // IDX_CHUNK
GROUPS = ROWS_PER_W // 16


@functools.partial(
    pl.kernel,
    out_type=(jax.ShapeDtypeStruct((B, DIM_S), jnp.float32),
              jax.ShapeDtypeStruct((B, DIM_T), jnp.float32)),
    mesh=plsc.VectorSubcoreMesh(core_axis_name="c", subcore_axis_name="s"),
    compiler_params=pltpu.CompilerParams(use_tc_tiling_on_sc=False),
    scratch_types=[
        pltpu.VMEM((ROWS_PER_W,), jnp.float32),
        pltpu.VMEM((ROWS_PER_W,), jnp.float32),
        pltpu.VMEM((N_CHUNKS, IDX_CHUNK), jnp.int32),
        pltpu.VMEM((N_CHUNKS, IDX_CHUNK), jnp.int32),
        pltpu.VMEM((ROWS_PER_W, DIM_S), jnp.float32),
        pltpu.VMEM((ROWS_PER_W, DIM_T), jnp.float32),
        pltpu.SemaphoreType.DMA,
    ],
)
def _sc_embed(sid_hbm, tid_hbm, w_s_hbm, w_t_hbm, out_s_hbm, out_t_hbm,
              sid_v, tid_v, idx_s_v, idx_t_v, emb_s_v, emb_t_v, sem):
    wid = lax.axis_index("s") * NC + lax.axis_index("c")
    base = wid * ROWS_PER_W

    pltpu.sync_copy(sid_hbm.at[pl.ds(base, ROWS_PER_W)], sid_v)
    pltpu.sync_copy(tid_hbm.at[pl.ds(base, ROWS_PER_W)], tid_v)

    for g in range(GROUPS):
        s_ids = sid_v[pl.ds(g * 16, 16)].astype(jnp.int32)
        t_ids = tid_v[pl.ds(g * 16, 16)].astype(jnp.int32)
        j, off = divmod(g * 16, IDX_CHUNK)
        idx_s_v[j, pl.ds(off, 16)] = s_ids
        idx_t_v[j, pl.ds(off, 16)] = t_ids

    copies = []
    for j in range(N_CHUNKS):
        rows_j = pl.ds(j * IDX_CHUNK, IDX_CHUNK)
        copies.append(pltpu.async_copy(
            w_s_hbm.at[idx_s_v.at[j]], emb_s_v.at[rows_j], sem))
        copies.append(pltpu.async_copy(
            w_t_hbm.at[idx_t_v.at[j]], emb_t_v.at[rows_j], sem))
    for c in copies:
        c.wait()

    pltpu.sync_copy(emb_s_v, out_s_hbm.at[pl.ds(base, ROWS_PER_W)])
    pltpu.sync_copy(emb_t_v, out_t_hbm.at[pl.ds(base, ROWS_PER_W)])


def kernel(x, W_s, W_t):
    emb_s, emb_t = _sc_embed(x[:, F_DENSE], x[:, F_DENSE + 1], W_s, W_t)
    return jnp.concatenate((x[:, :F_DENSE], emb_s, emb_t), axis=1)
